# baseline (speedup 1.0000x reference)
import functools

import jax
import jax.numpy as jnp
from jax import lax
from jax.experimental import pallas as pl
from jax.experimental.pallas import tpu as pltpu
from jax.experimental.pallas import tpu_sc as plsc

N = 10000
E = 320000
NFEAT = 128
NHID = 128
NCLASS = 64

NC = 2
NS = 16
NW = NC * NS
EPW = E // NW
CHUNK = 128
NBUF = 4
NCHUNK = 80
EPWP = NCHUNK * CHUNK
NGRP = NCHUNK // 4
NPAD = 10240
ROWS_PT = NPAD // NS
ZROWS = 128
D = 64


def _make_sc_segment_sum():
    mesh = plsc.VectorSubcoreMesh(core_axis_name="c", subcore_axis_name="s")

    @functools.partial(
        pl.kernel,
        out_type=jax.ShapeDtypeStruct((NC, NPAD, D), jnp.float32),
        mesh=mesh,
        compiler_params=pltpu.CompilerParams(use_tc_tiling_on_sc=False),
        scratch_types=[
            pltpu.VMEM((NCHUNK, CHUNK), jnp.int32),
            pltpu.VMEM((NCHUNK, CHUNK), jnp.int32),
            pltpu.VMEM((NBUF, CHUNK, D), jnp.float32),
            pltpu.VMEM((ZROWS, D), jnp.float32),
            pltpu.VMEM_SHARED((NPAD, D), jnp.float32),
            pltpu.SemaphoreType.DMA,
            pltpu.SemaphoreType.DMA,
        ],
    )
    def seg_sum(feat_hbm, src_hbm, dst_hbm, out_hbm, src_v, dst_v, rows_v,
                zero_v, acc_sh, gsem, ssem):
        cid = lax.axis_index("c")
        sid = lax.axis_index("s")
        wid = cid * NS + sid

        pltpu.sync_copy(src_hbm.at[wid], src_v)
        pltpu.sync_copy(dst_hbm.at[wid], dst_v)

        zvec = jnp.zeros((16,), jnp.float32)

        def zrow(r, _):
            for u in range(D // 16):
                zero_v[r, pl.ds(u * 16, 16)] = zvec
            return 0


        def gather(ci, buf):
            pltpu.async_copy(feat_hbm.at[src_v.at[ci]], rows_v.at[buf], gsem)

        def gwait(ci, buf):
            pltpu.make_async_copy(
                feat_hbm.at[src_v.at[ci]], rows_v.at[buf], gsem).wait()

        def scat(ci, buf):
            pltpu.async_copy(rows_v.at[buf], acc_sh.at[dst_v.at[ci]], ssem,
                             add=True)

        def swait(ci, buf):
            pltpu.make_async_copy(rows_v.at[buf], acc_sh.at[dst_v.at[ci]],
                                  ssem).wait()

        for b in range(NBUF):
            gather(b, b)

        lax.fori_loop(0, ZROWS, zrow, 0)
        base = sid * ROWS_PT
        for t in range(ROWS_PT // ZROWS):
            pltpu.sync_copy(zero_v, acc_sh.at[pl.ds(base + t * ZROWS, ZROWS)])
        plsc.subcore_barrier()

        for c in range(2):
            gwait(c, c)
            scat(c, c)

        def step(c, _):
            bfree = lax.rem(c + 2, NBUF)
            swait(c - 2, bfree)
            gather(c + 2, bfree)
            b = lax.rem(c, NBUF)
            gwait(c, b)
            scat(c, b)
            return 0

        lax.fori_loop(2, NCHUNK - 2, step, 0)
        for c in range(NCHUNK - 2, NCHUNK):
            b = c % NBUF
            swait(c - 2, (c + 2) % NBUF)
            gwait(c, b)
            scat(c, b)
        for c in range(NCHUNK - 2, NCHUNK):
            swait(c, c % NBUF)

        plsc.subcore_barrier()
        pltpu.sync_copy(acc_sh.at[pl.ds(base, ROWS_PT)],
                        out_hbm.at[cid, pl.ds(base, ROWS_PT)])

    return seg_sum


_seg_sum = _make_sc_segment_sum()

_BR = 2000
_BR2 = 1024
NPK = NPAD // 2


def _tc_matmul(x, w):
    n, k = x.shape
    m = w.shape[1]

    def body(x_ref, w_ref, o_ref):
        o_ref[...] = jnp.dot(x_ref[...], w_ref[...],
                             preferred_element_type=jnp.float32)

    return pl.pallas_call(
        body,
        grid=(n // _BR,),
        in_specs=[pl.BlockSpec((_BR, k), lambda i: (i, 0)),
                  pl.BlockSpec((k, m), lambda i: (0, 0))],
        out_specs=pl.BlockSpec((_BR, m), lambda i: (i, 0)),
        out_shape=jax.ShapeDtypeStruct((NPAD, m), jnp.float32),
    )(x, w)


def _tc_relu_matmul(pa, pb, ba, bb, w):
    def body(pa_ref, pb_ref, ba_ref, bb_ref, w_ref, o_ref):
        qa = jnp.maximum(pa_ref[0] + pa_ref[1] + ba_ref[...], 0.0)
        qb = jnp.maximum(pb_ref[0] + pb_ref[1] + bb_ref[...], 0.0)
        wa = w_ref[:64]
        wb = w_ref[64:]
        out_e = (jnp.dot(qa[:, :64], wa, preferred_element_type=jnp.float32)
                 + jnp.dot(qb[:, :64], wb, preferred_element_type=jnp.float32))
        out_o = (jnp.dot(qa[:, 64:], wa, preferred_element_type=jnp.float32)
                 + jnp.dot(qb[:, 64:], wb, preferred_element_type=jnp.float32))
        o_ref[...] = jnp.concatenate([out_e, out_o], axis=1)

    return pl.pallas_call(
        body,
        grid=(NPK // _BR2,),
        in_specs=[pl.BlockSpec((2, _BR2, 128), lambda i: (0, i, 0)),
                  pl.BlockSpec((2, _BR2, 128), lambda i: (0, i, 0)),
                  pl.BlockSpec((1, 128), lambda i: (0, 0)),
                  pl.BlockSpec((1, 128), lambda i: (0, 0)),
                  pl.BlockSpec((128, 64), lambda i: (0, 0))],
        out_specs=pl.BlockSpec((_BR2, 128), lambda i: (i, 0)),
        out_shape=jax.ShapeDtypeStruct((NPAD, 128), jnp.float32),
    )(pa, pb, ba, bb, w)


def _tc_bias_log_softmax(q, b):
    npk2 = N // 2

    def half_lsm(z):
        zmax = jnp.max(z, axis=1, keepdims=True)
        e = jnp.exp(z - zmax)
        ssum = jnp.sum(e, axis=1, keepdims=True)
        return z - zmax - jnp.log(ssum)

    def body(q_ref, b_ref, o_ref):
        z = q_ref[0] + q_ref[1] + b_ref[...]
        o_ref[...] = jnp.concatenate(
            [half_lsm(z[:, :64]), half_lsm(z[:, 64:])], axis=1)

    return pl.pallas_call(
        body,
        grid=(1,),
        in_specs=[pl.BlockSpec((2, npk2, 128), lambda i: (0, 0, 0)),
                  pl.BlockSpec((1, 128), lambda i: (0, 0))],
        out_specs=pl.BlockSpec((npk2, 128), lambda i: (0, 0)),
        out_shape=jax.ShapeDtypeStruct((npk2, 128), jnp.float32),
    )(q, b)


def kernel(x, edge_index, W1, b1, W2, b2):
    e = edge_index.reshape(2, NW, EPW)
    pad = EPWP - EPW
    wids = jnp.arange(NW, dtype=jnp.int32)
    src_rows = (wids * 311) % N
    dst_rows = N + (wids * 7) % (NPAD - N)
    src_pad = jnp.broadcast_to(src_rows[None, :, None], (1, NW, pad))
    dst_pad = jnp.broadcast_to(dst_rows[None, :, None], (1, NW, pad))
    ed = jnp.concatenate(
        [e, jnp.concatenate([src_pad, dst_pad], axis=0)], axis=2)
    src_c = ed[0].reshape(NW, NCHUNK, CHUNK)
    dst_c = ed[1].reshape(NW, NCHUNK, CHUNK)
    src_a = 2 * src_c
    src_b = src_a + 1

    s1 = _tc_matmul(x, W1)
    s1v = s1.reshape(2 * NPAD, 64)
    p1a = _seg_sum(s1v, src_a, dst_c)
    p1b = _seg_sum(s1v, src_b, dst_c)
    b1a = jnp.concatenate([b1[:64], b1[:64]]).reshape(1, 128)
    b1b = jnp.concatenate([b1[64:], b1[64:]]).reshape(1, 128)
    s2 = _tc_relu_matmul(p1a.reshape(2, NPK, 128), p1b.reshape(2, NPK, 128),
                         b1a, b1b, W2)
    p2 = _seg_sum(s2.reshape(2 * NPAD, 64), src_c, dst_c)
    b2c = jnp.concatenate([b2, b2]).reshape(1, 128)
    out = _tc_bias_log_softmax(p2.reshape(2, NPK, 128), b2c)
    return out.reshape(N, NCLASS)


# speedup vs baseline: 1.0051x; 1.0051x over previous
import functools

import jax
import jax.numpy as jnp
from jax import lax
from jax.experimental import pallas as pl
from jax.experimental.pallas import tpu as pltpu
from jax.experimental.pallas import tpu_sc as plsc

N = 10000
E = 320000
NFEAT = 128
NHID = 128
NCLASS = 64

NC = 2
NS = 16
NW = NC * NS
EPW = E // NW
CHUNK = 128
NBUF = 4
NCHUNK = 80
EPWP = NCHUNK * CHUNK
NGRP = NCHUNK // 4
NPAD = 10240
ROWS_PT = NPAD // NS
ZROWS = 128
D = 64


def _make_sc_segment_sum():
    mesh = plsc.VectorSubcoreMesh(core_axis_name="c", subcore_axis_name="s")

    @functools.partial(
        pl.kernel,
        out_type=jax.ShapeDtypeStruct((NC, NPAD, D), jnp.float32),
        mesh=mesh,
        compiler_params=pltpu.CompilerParams(use_tc_tiling_on_sc=False),
        scratch_types=[
            pltpu.VMEM((NCHUNK, CHUNK), jnp.int32),
            pltpu.VMEM((NCHUNK, CHUNK), jnp.int32),
            pltpu.VMEM((NBUF, CHUNK, D), jnp.float32),
            pltpu.VMEM((ZROWS, D), jnp.float32),
            pltpu.VMEM_SHARED((NPAD, D), jnp.float32),
            pltpu.SemaphoreType.DMA,
            pltpu.SemaphoreType.DMA,
        ],
    )
    def seg_sum(feat_hbm, edge_hbm, out_hbm, src_v, dst_v, rows_v, zero_v,
                acc_sh, gsem, ssem):
        cid = lax.axis_index("c")
        sid = lax.axis_index("s")
        wid = cid * NS + sid

        pltpu.sync_copy(edge_hbm.at[0, wid], src_v)
        pltpu.sync_copy(edge_hbm.at[1, wid], dst_v)

        zvec = jnp.zeros((16,), jnp.float32)

        def zrow(r, _):
            for u in range(D // 16):
                zero_v[r, pl.ds(u * 16, 16)] = zvec
            return 0


        def gather(ci, buf):
            pltpu.async_copy(feat_hbm.at[src_v.at[ci]], rows_v.at[buf], gsem)

        def gwait(ci, buf):
            pltpu.make_async_copy(
                feat_hbm.at[src_v.at[ci]], rows_v.at[buf], gsem).wait()

        def scat(ci, buf):
            pltpu.async_copy(rows_v.at[buf], acc_sh.at[dst_v.at[ci]], ssem,
                             add=True)

        def swait(ci, buf):
            pltpu.make_async_copy(rows_v.at[buf], acc_sh.at[dst_v.at[ci]],
                                  ssem).wait()

        for b in range(NBUF):
            gather(b, b)

        lax.fori_loop(0, ZROWS, zrow, 0)
        base = sid * ROWS_PT
        for t in range(ROWS_PT // ZROWS):
            pltpu.sync_copy(zero_v, acc_sh.at[pl.ds(base + t * ZROWS, ZROWS)])
        plsc.subcore_barrier()

        for c in range(2):
            gwait(c, c)
            scat(c, c)

        def step(c, _):
            bfree = lax.rem(c + 2, NBUF)
            swait(c - 2, bfree)
            gather(c + 2, bfree)
            b = lax.rem(c, NBUF)
            gwait(c, b)
            scat(c, b)
            return 0

        lax.fori_loop(2, NCHUNK - 2, step, 0)
        for c in range(NCHUNK - 2, NCHUNK):
            b = c % NBUF
            swait(c - 2, (c + 2) % NBUF)
            gwait(c, b)
            scat(c, b)
        for c in range(NCHUNK - 2, NCHUNK):
            swait(c, c % NBUF)

        plsc.subcore_barrier()
        pltpu.sync_copy(acc_sh.at[pl.ds(base, ROWS_PT)],
                        out_hbm.at[cid, pl.ds(base, ROWS_PT)])

    return seg_sum


_seg_sum = _make_sc_segment_sum()

_BR = 2000
_BR2 = 1024
NPK = NPAD // 2


def _tc_matmul(x, w):
    n, k = x.shape
    m = w.shape[1]

    def body(x_ref, w_ref, o_ref):
        o_ref[...] = jnp.dot(x_ref[...], w_ref[...],
                             preferred_element_type=jnp.float32)

    return pl.pallas_call(
        body,
        grid=(n // _BR,),
        in_specs=[pl.BlockSpec((_BR, k), lambda i: (i, 0)),
                  pl.BlockSpec((k, m), lambda i: (0, 0))],
        out_specs=pl.BlockSpec((_BR, m), lambda i: (i, 0)),
        out_shape=jax.ShapeDtypeStruct((NPAD, m), jnp.float32),
    )(x, w)


def _tc_relu_matmul(pa, pb, ba, bb, w):
    def body(pa_ref, pb_ref, ba_ref, bb_ref, w_ref, o_ref):
        qa = jnp.maximum(pa_ref[0] + pa_ref[1] + ba_ref[...], 0.0)
        qb = jnp.maximum(pb_ref[0] + pb_ref[1] + bb_ref[...], 0.0)
        wa = w_ref[:64]
        wb = w_ref[64:]
        out_e = (jnp.dot(qa[:, :64], wa, preferred_element_type=jnp.float32)
                 + jnp.dot(qb[:, :64], wb, preferred_element_type=jnp.float32))
        out_o = (jnp.dot(qa[:, 64:], wa, preferred_element_type=jnp.float32)
                 + jnp.dot(qb[:, 64:], wb, preferred_element_type=jnp.float32))
        o_ref[...] = jnp.concatenate([out_e, out_o], axis=1)

    return pl.pallas_call(
        body,
        grid=(NPK // _BR2,),
        in_specs=[pl.BlockSpec((2, _BR2, 128), lambda i: (0, i, 0)),
                  pl.BlockSpec((2, _BR2, 128), lambda i: (0, i, 0)),
                  pl.BlockSpec((1, 128), lambda i: (0, 0)),
                  pl.BlockSpec((1, 128), lambda i: (0, 0)),
                  pl.BlockSpec((128, 64), lambda i: (0, 0))],
        out_specs=pl.BlockSpec((_BR2, 128), lambda i: (i, 0)),
        out_shape=jax.ShapeDtypeStruct((NPAD, 128), jnp.float32),
    )(pa, pb, ba, bb, w)


def _tc_bias_log_softmax(q, b):
    npk2 = N // 2

    def half_lsm(z):
        zmax = jnp.max(z, axis=1, keepdims=True)
        e = jnp.exp(z - zmax)
        ssum = jnp.sum(e, axis=1, keepdims=True)
        return z - zmax - jnp.log(ssum)

    def body(q_ref, b_ref, o_ref):
        z = q_ref[0] + q_ref[1] + b_ref[...]
        o_ref[...] = jnp.concatenate(
            [half_lsm(z[:, :64]), half_lsm(z[:, 64:])], axis=1)

    return pl.pallas_call(
        body,
        grid=(1,),
        in_specs=[pl.BlockSpec((2, npk2, 128), lambda i: (0, 0, 0)),
                  pl.BlockSpec((1, 128), lambda i: (0, 0))],
        out_specs=pl.BlockSpec((npk2, 128), lambda i: (0, 0)),
        out_shape=jax.ShapeDtypeStruct((npk2, 128), jnp.float32),
    )(q, b)


def kernel(x, edge_index, W1, b1, W2, b2):
    e = edge_index.reshape(2, NW, EPW)
    pad = EPWP - EPW
    wids = jnp.arange(NW, dtype=jnp.int32)
    src_rows = (wids * 311) % N
    dst_rows = N + (wids * 7) % (NPAD - N)
    src_pad = jnp.broadcast_to(src_rows[None, :, None], (1, NW, pad))
    dst_pad = jnp.broadcast_to(dst_rows[None, :, None], (1, NW, pad))
    ed = jnp.concatenate(
        [e, jnp.concatenate([src_pad, dst_pad], axis=0)], axis=2)
    src, dst = ed[0], ed[1]
    edges_a = jnp.stack([2 * src, dst]).reshape(2, NW, NCHUNK, CHUNK)
    edges_b = jnp.stack([2 * src + 1, dst]).reshape(2, NW, NCHUNK, CHUNK)
    edges_c = ed.reshape(2, NW, NCHUNK, CHUNK)

    s1 = _tc_matmul(x, W1)
    s1v = s1.reshape(2 * NPAD, 64)
    p1a = _seg_sum(s1v, edges_a)
    p1b = _seg_sum(s1v, edges_b)
    b1a = jnp.concatenate([b1[:64], b1[:64]]).reshape(1, 128)
    b1b = jnp.concatenate([b1[64:], b1[64:]]).reshape(1, 128)
    s2 = _tc_relu_matmul(p1a.reshape(2, NPK, 128), p1b.reshape(2, NPK, 128),
                         b1a, b1b, W2)
    p2 = _seg_sum(s2.reshape(2 * NPAD, 64), edges_c)
    b2c = jnp.concatenate([b2, b2]).reshape(1, 128)
    out = _tc_bias_log_softmax(p2.reshape(2, NPK, 128), b2c)
    return out.reshape(N, NCLASS)


# speedup vs baseline: 1.0059x; 1.0008x over previous
import functools

import jax
import jax.numpy as jnp
from jax import lax
from jax.experimental import pallas as pl
from jax.experimental.pallas import tpu as pltpu
from jax.experimental.pallas import tpu_sc as plsc

N = 10000
E = 320000
NFEAT = 128
NHID = 128
NCLASS = 64

NC = 2
NS = 16
NW = NC * NS
EPW = E // NW
CHUNK = 128
NBUF = 4
NCHUNK = 80
EPWP = NCHUNK * CHUNK
NPAD = 10240
ROWS_PT = NPAD // NS
ZROWS = 128
D = 64


def _make_sc_segment_sum():
    mesh = plsc.VectorSubcoreMesh(core_axis_name="c", subcore_axis_name="s")

    @functools.partial(
        pl.kernel,
        out_type=jax.ShapeDtypeStruct((NC, NPAD, D), jnp.float32),
        mesh=mesh,
        compiler_params=pltpu.CompilerParams(use_tc_tiling_on_sc=False),
        scratch_types=[
            pltpu.VMEM((NCHUNK, CHUNK), jnp.int32),
            pltpu.VMEM((NCHUNK, CHUNK), jnp.int32),
            pltpu.VMEM((NBUF, CHUNK, D), jnp.float32),
            pltpu.VMEM((ZROWS, D), jnp.float32),
            pltpu.VMEM_SHARED((NPAD, D), jnp.float32),
            pltpu.SemaphoreType.DMA,
            pltpu.SemaphoreType.DMA,
        ],
    )
    def seg_sum(feat_hbm, edge_hbm, out_hbm, src_v, dst_v, rows_v, zero_v,
                acc_sh, gsem, ssem):
        cid = lax.axis_index("c")
        sid = lax.axis_index("s")
        wid = cid * NS + sid

        pltpu.sync_copy(edge_hbm.at[0, wid], src_v)
        pltpu.sync_copy(edge_hbm.at[1, wid], dst_v)

        zvec = jnp.zeros((16,), jnp.float32)

        def zrow(r, _):
            for u in range(D // 16):
                zero_v[r, pl.ds(u * 16, 16)] = zvec
            return 0


        def gather(ci, buf):
            pltpu.async_copy(feat_hbm.at[src_v.at[ci]], rows_v.at[buf], gsem)

        def gwait(ci, buf):
            pltpu.make_async_copy(
                feat_hbm.at[src_v.at[ci]], rows_v.at[buf], gsem).wait()

        def scat(ci, buf):
            pltpu.async_copy(rows_v.at[buf], acc_sh.at[dst_v.at[ci]], ssem,
                             add=True)

        def swait(ci, buf):
            pltpu.make_async_copy(rows_v.at[buf], acc_sh.at[dst_v.at[ci]],
                                  ssem).wait()

        for b in range(NBUF):
            gather(b, b)

        lax.fori_loop(0, ZROWS, zrow, 0)
        base = sid * ROWS_PT
        for t in range(ROWS_PT // ZROWS):
            pltpu.sync_copy(zero_v, acc_sh.at[pl.ds(base + t * ZROWS, ZROWS)])
        plsc.subcore_barrier()

        for c in range(2):
            gwait(c, c)
            scat(c, c)

        def step(c, _):
            bfree = lax.rem(c + 2, NBUF)
            swait(c - 2, bfree)
            gather(c + 2, bfree)
            b = lax.rem(c, NBUF)
            gwait(c, b)
            scat(c, b)
            return 0

        lax.fori_loop(2, NCHUNK - 2, step, 0)
        for c in range(NCHUNK - 2, NCHUNK):
            b = c % NBUF
            swait(c - 2, (c + 2) % NBUF)
            gwait(c, b)
            scat(c, b)
        for c in range(NCHUNK - 2, NCHUNK):
            swait(c, c % NBUF)

        plsc.subcore_barrier()
        pltpu.sync_copy(acc_sh.at[pl.ds(base, ROWS_PT)],
                        out_hbm.at[cid, pl.ds(base, ROWS_PT)])

    return seg_sum


_seg_sum = _make_sc_segment_sum()

_BR = 2000
_BR2 = 1024
NPK = NPAD // 2


def _tc_matmul(x, w):
    n, k = x.shape
    m = w.shape[1]

    def body(x_ref, w_ref, o_ref):
        o_ref[...] = jnp.dot(x_ref[...], w_ref[...],
                             preferred_element_type=jnp.float32)

    return pl.pallas_call(
        body,
        grid=(n // _BR,),
        in_specs=[pl.BlockSpec((_BR, k), lambda i: (i, 0)),
                  pl.BlockSpec((k, m), lambda i: (0, 0))],
        out_specs=pl.BlockSpec((_BR, m), lambda i: (i, 0)),
        out_shape=jax.ShapeDtypeStruct((NPAD, m), jnp.float32),
    )(x, w)


def _tc_relu_matmul(pa, pb, ba, bb, w):
    def body(pa_ref, pb_ref, ba_ref, bb_ref, w_ref, o_ref):
        qa = jnp.maximum(pa_ref[0] + pa_ref[1] + ba_ref[...], 0.0)
        qb = jnp.maximum(pb_ref[0] + pb_ref[1] + bb_ref[...], 0.0)
        wa = w_ref[:64]
        wb = w_ref[64:]
        out_e = (jnp.dot(qa[:, :64], wa, preferred_element_type=jnp.float32)
                 + jnp.dot(qb[:, :64], wb, preferred_element_type=jnp.float32))
        out_o = (jnp.dot(qa[:, 64:], wa, preferred_element_type=jnp.float32)
                 + jnp.dot(qb[:, 64:], wb, preferred_element_type=jnp.float32))
        o_ref[...] = jnp.concatenate([out_e, out_o], axis=1)

    return pl.pallas_call(
        body,
        grid=(NPK // _BR2,),
        in_specs=[pl.BlockSpec((2, _BR2, 128), lambda i: (0, i, 0)),
                  pl.BlockSpec((2, _BR2, 128), lambda i: (0, i, 0)),
                  pl.BlockSpec((1, 128), lambda i: (0, 0)),
                  pl.BlockSpec((1, 128), lambda i: (0, 0)),
                  pl.BlockSpec((128, 64), lambda i: (0, 0))],
        out_specs=pl.BlockSpec((_BR2, 128), lambda i: (i, 0)),
        out_shape=jax.ShapeDtypeStruct((NPAD, 128), jnp.float32),
    )(pa, pb, ba, bb, w)


def _tc_bias_log_softmax(q, b):
    npk2 = N // 2

    def half_lsm(z):
        zmax = jnp.max(z, axis=1, keepdims=True)
        e = jnp.exp(z - zmax)
        ssum = jnp.sum(e, axis=1, keepdims=True)
        return z - zmax - jnp.log(ssum)

    def body(q_ref, b_ref, o_ref):
        z = q_ref[0] + q_ref[1] + b_ref[...]
        o_ref[...] = jnp.concatenate(
            [half_lsm(z[:, :64]), half_lsm(z[:, 64:])], axis=1)

    return pl.pallas_call(
        body,
        grid=(1,),
        in_specs=[pl.BlockSpec((2, npk2, 128), lambda i: (0, 0, 0)),
                  pl.BlockSpec((1, 128), lambda i: (0, 0))],
        out_specs=pl.BlockSpec((npk2, 128), lambda i: (0, 0)),
        out_shape=jax.ShapeDtypeStruct((npk2, 128), jnp.float32),
    )(q, b)


def kernel(x, edge_index, W1, b1, W2, b2):
    e = edge_index.reshape(2, NW, EPW)
    pad = EPWP - EPW
    wids = jnp.arange(NW, dtype=jnp.int32)
    src_rows = (wids * 311) % N
    dst_rows = N + (wids * 7) % (NPAD - N)
    src_pad = jnp.broadcast_to(src_rows[None, :, None], (1, NW, pad))
    dst_pad = jnp.broadcast_to(dst_rows[None, :, None], (1, NW, pad))
    ed = jnp.concatenate(
        [e, jnp.concatenate([src_pad, dst_pad], axis=0)], axis=2)
    src, dst = ed[0], ed[1]
    edges_a = jnp.stack([2 * src, dst]).reshape(2, NW, NCHUNK, CHUNK)
    edges_b = jnp.stack([2 * src + 1, dst]).reshape(2, NW, NCHUNK, CHUNK)
    edges_c = ed.reshape(2, NW, NCHUNK, CHUNK)

    s1 = _tc_matmul(x, W1)
    s1v = s1.reshape(2 * NPAD, 64)
    p1a = _seg_sum(s1v, edges_a)
    p1b = _seg_sum(s1v, edges_b)
    b1a = jnp.concatenate([b1[:64], b1[:64]]).reshape(1, 128)
    b1b = jnp.concatenate([b1[64:], b1[64:]]).reshape(1, 128)
    s2 = _tc_relu_matmul(p1a.reshape(2, NPK, 128), p1b.reshape(2, NPK, 128),
                         b1a, b1b, W2)
    p2 = _seg_sum(s2.reshape(2 * NPAD, 64), edges_c)
    b2c = jnp.concatenate([b2, b2]).reshape(1, 128)
    out = _tc_bias_log_softmax(p2.reshape(2, NPK, 128), b2c)
    return out.reshape(N, NCLASS)
